# baseline (speedup 1.0000x reference)
import jax
import jax.numpy as jnp
from jax.experimental import pallas as pl

_BLOCK = 768

_DN_NT = (((1,), (1,)), ((), ()))
_DN_TT = (((0,), (0,)), ((), ()))


def _fused_body(img_ref, relt_ref, name_ref, chart_ref,
                wi_ref, bi_ref, wrt_ref, br_ref,
                wn_ref, bn_ref, wct_ref, bc_ref,
                oi_ref, or_ref, on_ref, oc_ref):
    f32 = jnp.float32
    oi_ref[...] = jax.lax.dot_general(
        img_ref[...], wi_ref[...], _DN_NT,
        preferred_element_type=f32) + bi_ref[...]
    or_ref[...] = jax.lax.dot_general(
        relt_ref[...], wrt_ref[...], _DN_TT,
        preferred_element_type=f32) + br_ref[...]
    on_ref[...] = jax.lax.dot_general(
        name_ref[...], wn_ref[...], _DN_NT,
        preferred_element_type=f32) + bn_ref[...]
    oc_ref[...] = jax.lax.dot_general(
        chart_ref[...], wct_ref[...], _DN_TT,
        preferred_element_type=f32) + bc_ref[...]


def kernel(input_idx, adj, mask, img_features, rel_features, name_features,
           char_features, W_img, b_img, W_rel, b_rel, W_name, b_name,
           W_char, b_char):
    n = img_features.shape[0]
    b = _BLOCK
    grid = (pl.cdiv(n, b),)

    rel_t = rel_features.T
    char_t = char_features.T
    wr_t = W_rel.T
    wc_t = W_char.T

    bi = b_img.reshape(1, -1)
    br = b_rel.reshape(1, -1)
    bn = b_name.reshape(1, -1)
    bc = b_char.reshape(1, -1)

    row_spec = lambda k: pl.BlockSpec((b, k), lambda i: (i, 0))
    col_spec = lambda k: pl.BlockSpec((k, b), lambda i: (0, i))
    full_spec = lambda a: pl.BlockSpec(a.shape, lambda i: (0,) * a.ndim)

    out_shapes = (
        jax.ShapeDtypeStruct((n, 512), jnp.float32),
        jax.ShapeDtypeStruct((n, 512), jnp.float32),
        jax.ShapeDtypeStruct((n, 256), jnp.float32),
        jax.ShapeDtypeStruct((n, 256), jnp.float32),
    )

    return pl.pallas_call(
        _fused_body,
        grid=grid,
        in_specs=[
            row_spec(2048), col_spec(1000), row_spec(512), col_spec(100),
            full_spec(W_img), full_spec(bi), full_spec(wr_t), full_spec(br),
            full_spec(W_name), full_spec(bn), full_spec(wc_t), full_spec(bc),
        ],
        out_specs=[
            row_spec(512), row_spec(512), row_spec(256), row_spec(256),
        ],
        out_shape=out_shapes,
    )(img_features, rel_t, name_features, char_t,
      W_img, bi, wr_t, br, W_name, bn, wc_t, bc)


# speedup vs baseline: 1.0265x; 1.0265x over previous
import jax
import jax.numpy as jnp
from jax.experimental import pallas as pl

_BLOCK = 1152

_DN_NT = (((1,), (1,)), ((), ()))
_DN_TT = (((0,), (0,)), ((), ()))


def _fused_body(img_ref, relt_ref, name_ref, chart_ref,
                wi_ref, bi_ref, wrt_ref, br_ref,
                wn_ref, bn_ref, wct_ref, bc_ref,
                oi_ref, or_ref, on_ref, oc_ref):
    f32 = jnp.float32
    oi_ref[...] = jax.lax.dot_general(
        img_ref[...], wi_ref[...], _DN_NT,
        preferred_element_type=f32) + bi_ref[...]
    or_ref[...] = jax.lax.dot_general(
        relt_ref[...], wrt_ref[...], _DN_TT,
        preferred_element_type=f32) + br_ref[...]
    on_ref[...] = jax.lax.dot_general(
        name_ref[...], wn_ref[...], _DN_NT,
        preferred_element_type=f32) + bn_ref[...]
    oc_ref[...] = jax.lax.dot_general(
        chart_ref[...], wct_ref[...], _DN_TT,
        preferred_element_type=f32) + bc_ref[...]


def kernel(input_idx, adj, mask, img_features, rel_features, name_features,
           char_features, W_img, b_img, W_rel, b_rel, W_name, b_name,
           W_char, b_char):
    n = img_features.shape[0]
    b = _BLOCK
    grid = (pl.cdiv(n, b),)

    rel_t = rel_features.T
    char_t = char_features.T
    wr_t = W_rel.T
    wc_t = W_char.T

    bi = b_img.reshape(1, -1)
    br = b_rel.reshape(1, -1)
    bn = b_name.reshape(1, -1)
    bc = b_char.reshape(1, -1)

    row_spec = lambda k: pl.BlockSpec((b, k), lambda i: (i, 0))
    col_spec = lambda k: pl.BlockSpec((k, b), lambda i: (0, i))
    full_spec = lambda a: pl.BlockSpec(a.shape, lambda i: (0,) * a.ndim)

    out_shapes = (
        jax.ShapeDtypeStruct((n, 512), jnp.float32),
        jax.ShapeDtypeStruct((n, 512), jnp.float32),
        jax.ShapeDtypeStruct((n, 256), jnp.float32),
        jax.ShapeDtypeStruct((n, 256), jnp.float32),
    )

    return pl.pallas_call(
        _fused_body,
        grid=grid,
        in_specs=[
            row_spec(2048), col_spec(1000), row_spec(512), col_spec(100),
            full_spec(W_img), full_spec(bi), full_spec(wr_t), full_spec(br),
            full_spec(W_name), full_spec(bn), full_spec(wc_t), full_spec(bc),
        ],
        out_specs=[
            row_spec(512), row_spec(512), row_spec(256), row_spec(256),
        ],
        out_shape=out_shapes,
    )(img_features, rel_t, name_features, char_t,
      W_img, bi, wr_t, br, W_name, bn, wc_t, bc)
